# Initial kernel scaffold; baseline (speedup 1.0000x reference)
import jax
import jax.numpy as jnp
from jax.experimental import pallas as pl


def kernel(x, W_qkv, b_qkv, W_proj, b_proj):
    raise NotImplementedError("write your pallas kernel here")


# speedup vs baseline: 1.4098x; 1.4098x over previous
import functools

import jax
import jax.numpy as jnp
from jax.experimental import pallas as pl
from jax.experimental.pallas import tpu as pltpu

H = 16
BQ = 512


def _qkv_body(x_ref, w_ref, b_ref, o_ref):
    x = x_ref[0]
    w = w_ref[0]
    o = jax.lax.dot_general(x, w, (((1,), (1,)), ((), ())),
                            preferred_element_type=jnp.float32)
    o_ref[0, 0] = o + b_ref[0]


def _attn_body(q_ref, k_ref, v_ref, wp_ref, bp_ref, o_ref, *, scale):
    q = q_ref[0, 0]
    k = k_ref[0, 0]
    v = v_ref[0, 0]
    s = jax.lax.dot_general(q, k, (((1,), (1,)), ((), ())),
                            preferred_element_type=jnp.float32) * scale
    m = jnp.max(s, axis=-1, keepdims=True)
    p = jnp.exp(s - m)
    l = jnp.sum(p, axis=-1, keepdims=True)
    o = jnp.dot(p, v, preferred_element_type=jnp.float32) / l
    acc = jnp.dot(o, wp_ref[0], preferred_element_type=jnp.float32)
    h = pl.program_id(2)

    @pl.when(h == 0)
    def _():
        o_ref[0] = acc + bp_ref[...]

    @pl.when(h != 0)
    def _():
        o_ref[0] = o_ref[0] + acc


def kernel(x, W_qkv, b_qkv, W_proj, b_proj):
    B, N, C = x.shape
    HD = C // H
    scale = HD ** -0.5

    Wr = W_qkv.reshape(3 * H, HD, C)
    br = b_qkv.reshape(3 * H, 1, HD)
    qkv = pl.pallas_call(
        _qkv_body,
        grid=(B, 3 * H),
        in_specs=[
            pl.BlockSpec((1, N, C), lambda b, g: (b, 0, 0)),
            pl.BlockSpec((1, HD, C), lambda b, g: (g, 0, 0)),
            pl.BlockSpec((1, 1, HD), lambda b, g: (g, 0, 0)),
        ],
        out_specs=pl.BlockSpec((1, 1, N, HD), lambda b, g: (b, g, 0, 0)),
        out_shape=jax.ShapeDtypeStruct((B, 3 * H, N, HD), jnp.float32),
        compiler_params=pltpu.CompilerParams(
            dimension_semantics=("parallel", "parallel")),
    )(x, Wr, br)

    WpT = W_proj.T.reshape(H, HD, C)
    bp = b_proj.reshape(1, C)
    out = pl.pallas_call(
        functools.partial(_attn_body, scale=scale),
        grid=(B, N // BQ, H),
        in_specs=[
            pl.BlockSpec((1, 1, BQ, HD), lambda b, qi, h: (b, h, qi, 0)),
            pl.BlockSpec((1, 1, N, HD), lambda b, qi, h: (b, H + h, 0, 0)),
            pl.BlockSpec((1, 1, N, HD), lambda b, qi, h: (b, 2 * H + h, 0, 0)),
            pl.BlockSpec((1, HD, C), lambda b, qi, h: (h, 0, 0)),
            pl.BlockSpec((1, C), lambda b, qi, h: (0, 0)),
        ],
        out_specs=pl.BlockSpec((1, BQ, C), lambda b, qi, h: (b, qi, 0)),
        out_shape=jax.ShapeDtypeStruct((B, N, C), jnp.float32),
        compiler_params=pltpu.CompilerParams(
            dimension_semantics=("parallel", "parallel", "arbitrary")),
    )(qkv, qkv, qkv, WpT, bp)
    return out
